# baseline (speedup 1.0000x reference)
import functools

import jax
import jax.numpy as jnp
from jax import lax
from jax.experimental import pallas as pl
from jax.experimental.pallas import tpu as pltpu

D3_A1 = 0.3385
D3_A2 = 2.883
D3_K1 = 16.0
D3_K3 = -4.0
D3_S6 = 1.0
D3_S8 = 0.9171
D3_MAXC2 = 25
EPS = 1e-10

LANES = 128
VMEM_LIMIT = 64 * 1024 * 1024


def _round_up(x, m):
    return ((x + m - 1) // m) * m


def _onehots(idx, nh):
    pt = idx.shape[-1]
    hi = idx >> 7
    lo = idx & 127
    ioh = lax.broadcasted_iota(jnp.int32, (nh, pt), 0)
    iol = lax.broadcasted_iota(jnp.int32, (LANES, pt), 0)
    a = jnp.where(hi == ioh, 1.0, 0.0)
    b = jnp.where(lo == iol, 1.0, 0.0)
    return a, b


def _sel(c, b):
    return jnp.sum(c * b, axis=0, keepdims=True)


def _rowsel(w, a):
    return lax.dot_general(w, a, dimension_numbers=(((1,), (0,)), ((), ())),
                           preferred_element_type=jnp.float32)


def _segsum(a, bvals, out_ref):
    out_ref[...] += lax.dot_general(
        a, bvals, dimension_numbers=(((1,), (1,)), ((), ())),
        preferred_element_type=jnp.float32)


def _k1_kernel(r_ref, ii_ref, ij_ref, w_ref, nc_ref, cls_ref, rp_ref,
               *, nh, maxz):
    @pl.when(pl.program_id(1) == 0)
    def _init():
        nc_ref[...] = jnp.zeros_like(nc_ref)

    ai, bi = _onehots(ii_ref[...], nh)
    aj, bj = _onehots(ij_ref[...], nh)
    w = w_ref[...]
    ci = _rowsel(w, ai)
    cj = _rowsel(w, aj)

    zi = _sel(ci[0:128], bi)
    zj = _sel(cj[0:128], bj)
    rco = _sel(ci[128:256], bi) + _sel(cj[128:256], bj)
    rp_ref[...] = _sel(ci[256:384], bi) * _sel(cj[256:384], bj)
    cls_ref[...] = (zi * maxz + zj + 0.5).astype(jnp.int32)

    rr = rco / r_ref[...]
    damp = 1.0 / (1.0 + jnp.exp(-D3_K1 * (rr - 1.0)))
    _segsum(ai, bi * damp, nc_ref)


def _k23_kernel(r_ref, ii_ref, ij_ref, rp_ref, tab_ref, nct_ref, out_ref,
                *, nh):
    @pl.when(pl.program_id(1) == 0)
    def _init():
        out_ref[...] = jnp.zeros_like(out_ref)

    ai, bi = _onehots(ii_ref[...], nh)
    aj, bj = _onehots(ij_ref[...], nh)
    nct = nct_ref[...]
    nci = _sel(_rowsel(nct, ai), bi)
    ncj = _sel(_rowsel(nct, aj), bj)

    cn0 = tab_ref[0:25]
    cn1 = tab_ref[32:57]
    cn2 = tab_ref[64:89]

    rdist = (cn1 - nci) ** 2 + (cn2 - ncj) ** 2
    logits = D3_K3 * rdist
    m = jnp.max(logits, axis=0, keepdims=True)
    w = jnp.exp(logits - m)
    wsum = jnp.sum(w, axis=0, keepdims=True)
    c6 = jnp.sum(w * cn0, axis=0, keepdims=True) / wsum

    r = r_ref[...]
    c8 = 3.0 * c6 * rp_ref[...]
    r2 = r * r
    r6 = r2 * r2 * r2
    r8 = r6 * r2
    tmp = D3_A1 * jnp.sqrt(c8 / (c6 + EPS) + EPS) + D3_A2
    tmp2 = tmp * tmp
    tmp6 = tmp2 * tmp2 * tmp2
    tmp8 = tmp6 * tmp2
    e = -0.5 * (D3_S6 * c6 / (r6 + tmp6) + D3_S8 * c8 / (r8 + tmp8))
    _segsum(ai, bi * e, out_ref)


def _tab_gather_kernel(cls_ref, tab3_ref, out_ref, *, pt, unroll):
    def body(o, carry):
        base = o * unroll
        for i in range(unroll):
            mi = base + i
            c = cls_ref[0, mi]
            out_ref[pl.ds(mi, 1)] = tab3_ref[pl.ds(c, 1)]
        return carry
    lax.fori_loop(0, pt // unroll, body, 0, unroll=False)


def kernel(Z, r, idx_i, idx_j, c6ab_flat, rcov, r2r4):
    N = Z.shape[0]
    P = r.shape[0]
    MAXZ = rcov.shape[0]

    N_pad = _round_up(N, LANES * LANES)
    NH = N_pad // LANES

    PT = 2048
    P_pad = _round_up(P, 2 * PT)
    n_half = P_pad // (2 * PT)
    pad_p = P_pad - P

    idx_i = idx_i.astype(jnp.int32)
    idx_j = idx_j.astype(jnp.int32)
    ii_row = jnp.pad(idx_i, (0, pad_p), constant_values=N_pad).reshape(1, P_pad)
    ij_row = jnp.pad(idx_j, (0, pad_p), constant_values=N_pad).reshape(1, P_pad)
    r_row = jnp.pad(r.astype(jnp.float32), (0, pad_p),
                    constant_values=1.0).reshape(1, P_pad)

    zf = jnp.pad(Z.astype(jnp.float32), (0, N_pad - N))
    rcovz = jnp.pad(rcov[Z].astype(jnp.float32), (0, N_pad - N))
    r2r4z = jnp.pad(r2r4[Z].astype(jnp.float32), (0, N_pad - N))
    w_tab = jnp.stack([zf, rcovz, r2r4z]).reshape(3, NH, LANES)
    w_tab = w_tab.transpose(0, 2, 1).reshape(3 * LANES, NH)

    pair_spec = pl.BlockSpec((1, PT), lambda c, p: (0, c * n_half + p))
    atom_out_spec = pl.BlockSpec((None, NH, LANES), lambda c, p: (c, 0, 0))
    params = pltpu.CompilerParams(
        dimension_semantics=("parallel", "arbitrary"),
        vmem_limit_bytes=VMEM_LIMIT)

    nc_parts, cls_row, rp_row = pl.pallas_call(
        functools.partial(_k1_kernel, nh=NH, maxz=MAXZ),
        out_shape=(jax.ShapeDtypeStruct((2, NH, LANES), jnp.float32),
                   jax.ShapeDtypeStruct((1, P_pad), jnp.int32),
                   jax.ShapeDtypeStruct((1, P_pad), jnp.float32)),
        grid=(2, n_half),
        in_specs=[pair_spec, pair_spec, pair_spec,
                  pl.BlockSpec((3 * LANES, NH), lambda c, p: (0, 0))],
        out_specs=(atom_out_spec, pair_spec, pair_spec),
        compiler_params=params,
    )(r_row, ii_row, ij_row, w_tab)

    nc_t = jnp.sum(nc_parts, axis=0).T

    tab_p = jnp.pad(c6ab_flat.reshape(3, D3_MAXC2, MAXZ * MAXZ),
                    ((0, 0), (0, 7), (0, 0))).reshape(96, MAXZ * MAXZ)
    tab3 = tab_p.T.reshape(MAXZ * MAXZ, 1, 96)
    tab_pm = pl.pallas_call(
        functools.partial(_tab_gather_kernel, pt=PT, unroll=16),
        out_shape=jax.ShapeDtypeStruct((P_pad, 1, 96), jnp.float32),
        grid=(2, n_half),
        in_specs=[pl.BlockSpec((1, PT), lambda c, p: (0, c * n_half + p),
                               memory_space=pltpu.SMEM),
                  pl.BlockSpec((MAXZ * MAXZ, 1, 96), lambda c, p: (0, 0, 0))],
        out_specs=pl.BlockSpec((PT, 1, 96),
                               lambda c, p: (c * n_half + p, 0, 0)),
        compiler_params=params,
    )(cls_row, tab3)
    tab = tab_pm.reshape(P_pad, 96)[:, :89].T

    e_parts = pl.pallas_call(
        functools.partial(_k23_kernel, nh=NH),
        out_shape=jax.ShapeDtypeStruct((2, NH, LANES), jnp.float32),
        grid=(2, n_half),
        in_specs=[pair_spec, pair_spec, pair_spec, pair_spec,
                  pl.BlockSpec((89, PT), lambda c, p: (0, c * n_half + p)),
                  pl.BlockSpec((LANES, NH), lambda c, p: (0, 0))],
        out_specs=atom_out_spec,
        compiler_params=params,
    )(r_row, ii_row, ij_row, rp_row, tab, nc_t)

    return jnp.sum(e_parts, axis=0).reshape(N_pad)[:N]


# speedup vs baseline: 1.1183x; 1.1183x over previous
import functools

import jax
import jax.numpy as jnp
from jax import lax
from jax.experimental import pallas as pl
from jax.experimental.pallas import tpu as pltpu

D3_A1 = 0.3385
D3_A2 = 2.883
D3_K1 = 16.0
D3_K3 = -4.0
D3_S6 = 1.0
D3_S8 = 0.9171
D3_MAXC2 = 25
EPS = 1e-10

LANES = 128
VMEM_LIMIT = 64 * 1024 * 1024


def _round_up(x, m):
    return ((x + m - 1) // m) * m


def _onehots(idx, nh):
    pt = idx.shape[-1]
    hi = idx >> 7
    lo = idx & 127
    ioh = lax.broadcasted_iota(jnp.int32, (nh, pt), 0)
    iol = lax.broadcasted_iota(jnp.int32, (LANES, pt), 0)
    a = jnp.where(hi == ioh, 1.0, 0.0)
    b = jnp.where(lo == iol, 1.0, 0.0)
    return a, b


def _sel(c, b):
    return jnp.sum(c * b, axis=0, keepdims=True)


def _rowsel(w, a):
    return lax.dot_general(w, a, dimension_numbers=(((1,), (0,)), ((), ())),
                           preferred_element_type=jnp.float32)


def _segsum(a, bvals, out_ref):
    out_ref[...] += lax.dot_general(
        a, bvals, dimension_numbers=(((1,), (1,)), ((), ())),
        preferred_element_type=jnp.float32)


def _k1_kernel(r_ref, ii_ref, ij_ref, w_ref, nc_ref, cls_ref, rp_ref,
               *, nh, maxz):
    @pl.when(pl.program_id(1) == 0)
    def _init():
        nc_ref[...] = jnp.zeros_like(nc_ref)

    ai, bi = _onehots(ii_ref[...], nh)
    aj, bj = _onehots(ij_ref[...], nh)
    w = w_ref[...]
    ci = _rowsel(w, ai)
    cj = _rowsel(w, aj)

    zi = _sel(ci[0:128], bi)
    zj = _sel(cj[0:128], bj)
    rco = _sel(ci[128:256], bi) + _sel(cj[128:256], bj)
    rp_ref[...] = _sel(ci[256:384], bi) * _sel(cj[256:384], bj)
    cls_ref[...] = (zi * maxz + zj + 0.5).astype(jnp.int32)

    rr = rco / r_ref[...]
    damp = 1.0 / (1.0 + jnp.exp(-D3_K1 * (rr - 1.0)))
    _segsum(ai, bi * damp, nc_ref)


def _k23_kernel(r_ref, ii_ref, ij_ref, rp_ref, tab_ref, nct_ref, out_ref,
                *, nh):
    @pl.when(pl.program_id(1) == 0)
    def _init():
        out_ref[...] = jnp.zeros_like(out_ref)

    ai, bi = _onehots(ii_ref[...], nh)
    aj, bj = _onehots(ij_ref[...], nh)
    nct = nct_ref[...]
    nci = _sel(_rowsel(nct, ai), bi)
    ncj = _sel(_rowsel(nct, aj), bj)

    cn0 = tab_ref[0:25]
    cn1 = tab_ref[32:57]
    cn2 = tab_ref[64:89]

    rdist = (cn1 - nci) ** 2 + (cn2 - ncj) ** 2
    logits = D3_K3 * rdist
    m = jnp.max(logits, axis=0, keepdims=True)
    w = jnp.exp(logits - m)
    wsum = jnp.sum(w, axis=0, keepdims=True)
    c6 = jnp.sum(w * cn0, axis=0, keepdims=True) / wsum

    r = r_ref[...]
    c8 = 3.0 * c6 * rp_ref[...]
    r2 = r * r
    r6 = r2 * r2 * r2
    r8 = r6 * r2
    tmp = D3_A1 * jnp.sqrt(c8 / (c6 + EPS) + EPS) + D3_A2
    tmp2 = tmp * tmp
    tmp6 = tmp2 * tmp2 * tmp2
    tmp8 = tmp6 * tmp2
    e = -0.5 * (D3_S6 * c6 / (r6 + tmp6) + D3_S8 * c8 / (r8 + tmp8))
    _segsum(ai, bi * e, out_ref)


def _tab_gather_kernel(cls_ref, tab3_ref, out_ref, *, pt, unroll):
    def body(o, carry):
        base = o * unroll
        for i in range(unroll):
            mi = base + i
            c = cls_ref[0, mi]
            out_ref[pl.ds(mi, 1)] = tab3_ref[pl.ds(c, 1)]
        return carry
    lax.fori_loop(0, pt // unroll, body, 0, unroll=False)


def kernel(Z, r, idx_i, idx_j, c6ab_flat, rcov, r2r4):
    N = Z.shape[0]
    P = r.shape[0]
    MAXZ = rcov.shape[0]

    N_pad = _round_up(N, LANES * LANES)
    NH = N_pad // LANES

    PT = 2048
    P_pad = _round_up(P, 2 * PT)
    n_half = P_pad // (2 * PT)
    pad_p = P_pad - P

    idx_i = idx_i.astype(jnp.int32)
    idx_j = idx_j.astype(jnp.int32)
    ii_row = jnp.pad(idx_i, (0, pad_p), constant_values=N_pad).reshape(1, P_pad)
    ij_row = jnp.pad(idx_j, (0, pad_p), constant_values=N_pad).reshape(1, P_pad)
    r_row = jnp.pad(r.astype(jnp.float32), (0, pad_p),
                    constant_values=1.0).reshape(1, P_pad)

    zf = jnp.pad(Z.astype(jnp.float32), (0, N_pad - N))
    rcovz = jnp.pad(rcov[Z].astype(jnp.float32), (0, N_pad - N))
    r2r4z = jnp.pad(r2r4[Z].astype(jnp.float32), (0, N_pad - N))
    w_tab = jnp.stack([zf, rcovz, r2r4z]).reshape(3, NH, LANES)
    w_tab = w_tab.transpose(0, 2, 1).reshape(3 * LANES, NH)

    pair_spec = pl.BlockSpec((1, PT), lambda c, p: (0, c * n_half + p))
    atom_out_spec = pl.BlockSpec((None, NH, LANES), lambda c, p: (c, 0, 0))
    params = pltpu.CompilerParams(
        dimension_semantics=("parallel", "arbitrary"),
        vmem_limit_bytes=VMEM_LIMIT)

    nc_parts, cls_row, rp_row = pl.pallas_call(
        functools.partial(_k1_kernel, nh=NH, maxz=MAXZ),
        out_shape=(jax.ShapeDtypeStruct((2, NH, LANES), jnp.float32),
                   jax.ShapeDtypeStruct((1, P_pad), jnp.int32),
                   jax.ShapeDtypeStruct((1, P_pad), jnp.float32)),
        grid=(2, n_half),
        in_specs=[pair_spec, pair_spec, pair_spec,
                  pl.BlockSpec((3 * LANES, NH), lambda c, p: (0, 0))],
        out_specs=(atom_out_spec, pair_spec, pair_spec),
        compiler_params=params,
    )(r_row, ii_row, ij_row, w_tab)

    nc_t = jnp.sum(nc_parts, axis=0).T

    tab_p = jnp.pad(c6ab_flat.reshape(3, D3_MAXC2, MAXZ * MAXZ),
                    ((0, 0), (0, 7), (0, 0))).reshape(96, MAXZ * MAXZ)
    tab3 = tab_p.T.reshape(MAXZ * MAXZ, 1, 96)
    tab_pm = pl.pallas_call(
        functools.partial(_tab_gather_kernel, pt=PT, unroll=32),
        out_shape=jax.ShapeDtypeStruct((P_pad, 1, 96), jnp.float32),
        grid=(2, n_half),
        in_specs=[pl.BlockSpec((1, PT), lambda c, p: (0, c * n_half + p),
                               memory_space=pltpu.SMEM),
                  pl.BlockSpec((MAXZ * MAXZ, 1, 96), lambda c, p: (0, 0, 0))],
        out_specs=pl.BlockSpec((PT, 1, 96),
                               lambda c, p: (c * n_half + p, 0, 0)),
        compiler_params=params,
    )(cls_row, tab3)
    tab = tab_pm.reshape(P_pad, 96).T

    e_parts = pl.pallas_call(
        functools.partial(_k23_kernel, nh=NH),
        out_shape=jax.ShapeDtypeStruct((2, NH, LANES), jnp.float32),
        grid=(2, n_half),
        in_specs=[pair_spec, pair_spec, pair_spec, pair_spec,
                  pl.BlockSpec((96, PT), lambda c, p: (0, c * n_half + p)),
                  pl.BlockSpec((LANES, NH), lambda c, p: (0, 0))],
        out_specs=atom_out_spec,
        compiler_params=params,
    )(r_row, ii_row, ij_row, rp_row, tab, nc_t)

    return jnp.sum(e_parts, axis=0).reshape(N_pad)[:N]


# speedup vs baseline: 1.1436x; 1.0226x over previous
import functools

import jax
import jax.numpy as jnp
from jax import lax
from jax.experimental import pallas as pl
from jax.experimental.pallas import tpu as pltpu

D3_A1 = 0.3385
D3_A2 = 2.883
D3_K1 = 16.0
D3_K3 = -4.0
D3_S6 = 1.0
D3_S8 = 0.9171
D3_MAXC2 = 25
EPS = 1e-10

LANES = 128
VMEM_LIMIT = 64 * 1024 * 1024


def _round_up(x, m):
    return ((x + m - 1) // m) * m


def _onehots(idx, nh):
    pt = idx.shape[-1]
    hi = idx >> 7
    lo = idx & 127
    ioh = lax.broadcasted_iota(jnp.int32, (nh, pt), 0)
    iol = lax.broadcasted_iota(jnp.int32, (LANES, pt), 0)
    a = jnp.where(hi == ioh, 1.0, 0.0)
    b = jnp.where(lo == iol, 1.0, 0.0)
    return a, b


def _sel(c, b):
    return jnp.sum(c * b, axis=0, keepdims=True)


def _rowsel(w, a):
    return lax.dot_general(w, a, dimension_numbers=(((1,), (0,)), ((), ())),
                           preferred_element_type=jnp.float32)


def _segsum(a, bvals, out_ref):
    out_ref[...] += lax.dot_general(
        a, bvals, dimension_numbers=(((1,), (1,)), ((), ())),
        preferred_element_type=jnp.float32)


def _k1_kernel(r_ref, ii_ref, ij_ref, w_ref, nc_ref, cls_ref, rp_ref,
               *, nh, maxz):
    @pl.when(pl.program_id(1) == 0)
    def _init():
        nc_ref[...] = jnp.zeros_like(nc_ref)

    ai, bi = _onehots(ii_ref[...], nh)
    aj, bj = _onehots(ij_ref[...], nh)
    w = w_ref[...]
    ci = _rowsel(w, ai)
    cj = _rowsel(w, aj)

    zi = _sel(ci[0:128], bi)
    zj = _sel(cj[0:128], bj)
    rco = _sel(ci[128:256], bi) + _sel(cj[128:256], bj)
    rp_ref[...] = _sel(ci[256:384], bi) * _sel(cj[256:384], bj)
    cls_ref[...] = (zi * maxz + zj + 0.5).astype(jnp.int32)

    rr = rco / r_ref[...]
    damp = 1.0 / (1.0 + jnp.exp(-D3_K1 * (rr - 1.0)))
    _segsum(ai, bi * damp, nc_ref)


def _k23_kernel(r_ref, ii_ref, ij_ref, rp_ref, tab_ref, nct_ref, out_ref,
                *, nh):
    @pl.when(pl.program_id(1) == 0)
    def _init():
        out_ref[...] = jnp.zeros_like(out_ref)

    ai, bi = _onehots(ii_ref[...], nh)
    aj, bj = _onehots(ij_ref[...], nh)
    nct = nct_ref[...]
    nci = _sel(_rowsel(nct, ai), bi)
    ncj = _sel(_rowsel(nct, aj), bj)

    cn0 = tab_ref[0:25]
    cn1 = tab_ref[32:57]
    cn2 = tab_ref[64:89]

    rdist = (cn1 - nci) ** 2 + (cn2 - ncj) ** 2
    logits = D3_K3 * rdist
    m = jnp.max(logits, axis=0, keepdims=True)
    w = jnp.exp(logits - m)
    wsum = jnp.sum(w, axis=0, keepdims=True)
    c6 = jnp.sum(w * cn0, axis=0, keepdims=True) / wsum

    r = r_ref[...]
    c8 = 3.0 * c6 * rp_ref[...]
    r2 = r * r
    r6 = r2 * r2 * r2
    r8 = r6 * r2
    tmp = D3_A1 * jnp.sqrt(c8 / (c6 + EPS) + EPS) + D3_A2
    tmp2 = tmp * tmp
    tmp6 = tmp2 * tmp2 * tmp2
    tmp8 = tmp6 * tmp2
    e = -0.5 * (D3_S6 * c6 / (r6 + tmp6) + D3_S8 * c8 / (r8 + tmp8))
    _segsum(ai, bi * e, out_ref)


def _tab_gather_kernel(cls_ref, tab3_ref, out_ref, *, pt, unroll):
    def body(o, carry):
        base = o * unroll
        for i in range(unroll):
            mi = base + i
            c = cls_ref[0, mi]
            out_ref[pl.ds(mi, 1)] = tab3_ref[pl.ds(c, 1)]
        return carry
    lax.fori_loop(0, pt // unroll, body, 0, unroll=False)


def kernel(Z, r, idx_i, idx_j, c6ab_flat, rcov, r2r4):
    N = Z.shape[0]
    P = r.shape[0]
    MAXZ = rcov.shape[0]

    N_pad = _round_up(N, LANES * LANES)
    NH = N_pad // LANES

    PT = 2048
    P_pad = _round_up(P, 2 * PT)
    n_half = P_pad // (2 * PT)
    pad_p = P_pad - P

    idx_i = idx_i.astype(jnp.int32)
    idx_j = idx_j.astype(jnp.int32)
    ii_row = jnp.pad(idx_i, (0, pad_p), constant_values=N_pad).reshape(1, P_pad)
    ij_row = jnp.pad(idx_j, (0, pad_p), constant_values=N_pad).reshape(1, P_pad)
    r_row = jnp.pad(r.astype(jnp.float32), (0, pad_p),
                    constant_values=1.0).reshape(1, P_pad)

    zf = jnp.pad(Z.astype(jnp.float32), (0, N_pad - N))
    rcovz = jnp.pad(rcov[Z].astype(jnp.float32), (0, N_pad - N))
    r2r4z = jnp.pad(r2r4[Z].astype(jnp.float32), (0, N_pad - N))
    w_tab = jnp.stack([zf, rcovz, r2r4z]).reshape(3, NH, LANES)
    w_tab = w_tab.transpose(0, 2, 1).reshape(3 * LANES, NH)

    pair_spec = pl.BlockSpec((1, PT), lambda c, p: (0, c * n_half + p))
    atom_out_spec = pl.BlockSpec((None, NH, LANES), lambda c, p: (c, 0, 0))
    params = pltpu.CompilerParams(
        dimension_semantics=("parallel", "arbitrary"),
        vmem_limit_bytes=VMEM_LIMIT)

    nc_parts, cls_row, rp_row = pl.pallas_call(
        functools.partial(_k1_kernel, nh=NH, maxz=MAXZ),
        out_shape=(jax.ShapeDtypeStruct((2, NH, LANES), jnp.float32),
                   jax.ShapeDtypeStruct((1, P_pad), jnp.int32),
                   jax.ShapeDtypeStruct((1, P_pad), jnp.float32)),
        grid=(2, n_half),
        in_specs=[pair_spec, pair_spec, pair_spec,
                  pl.BlockSpec((3 * LANES, NH), lambda c, p: (0, 0))],
        out_specs=(atom_out_spec, pair_spec, pair_spec),
        compiler_params=params,
    )(r_row, ii_row, ij_row, w_tab)

    nc_t = jnp.sum(nc_parts, axis=0).T

    tab_p = jnp.pad(c6ab_flat.reshape(3, D3_MAXC2, MAXZ * MAXZ),
                    ((0, 0), (0, 7), (0, 0))).reshape(96, MAXZ * MAXZ)
    tab3 = tab_p.T.reshape(MAXZ * MAXZ, 1, 96)
    tab_pm = pl.pallas_call(
        functools.partial(_tab_gather_kernel, pt=PT, unroll=64),
        out_shape=jax.ShapeDtypeStruct((P_pad, 1, 96), jnp.float32),
        grid=(2, n_half),
        in_specs=[pl.BlockSpec((1, PT), lambda c, p: (0, c * n_half + p),
                               memory_space=pltpu.SMEM),
                  pl.BlockSpec((MAXZ * MAXZ, 1, 96), lambda c, p: (0, 0, 0))],
        out_specs=pl.BlockSpec((PT, 1, 96),
                               lambda c, p: (c * n_half + p, 0, 0)),
        compiler_params=params,
    )(cls_row, tab3)
    tab = tab_pm.reshape(P_pad, 96).T

    e_parts = pl.pallas_call(
        functools.partial(_k23_kernel, nh=NH),
        out_shape=jax.ShapeDtypeStruct((2, NH, LANES), jnp.float32),
        grid=(2, n_half),
        in_specs=[pair_spec, pair_spec, pair_spec, pair_spec,
                  pl.BlockSpec((96, PT), lambda c, p: (0, c * n_half + p)),
                  pl.BlockSpec((LANES, NH), lambda c, p: (0, 0))],
        out_specs=atom_out_spec,
        compiler_params=params,
    )(r_row, ii_row, ij_row, rp_row, tab, nc_t)

    return jnp.sum(e_parts, axis=0).reshape(N_pad)[:N]
